# baseline (speedup 1.0000x reference)
import jax
import jax.numpy as jnp
from jax import lax
from jax.experimental import pallas as pl
from jax.experimental.pallas import tpu as pltpu
from jax.experimental.pallas import tpu_sc as plsc

N = 10000
E = 320000
D = 128

NC = 2
NS = 16
B = 80
CPT = 128
NCH = NC * NS * CPT
EPAD = NCH * B
ROWS_PER_TILE = 624
TAIL_ROWS = N - NS * ROWS_PER_TILE


def _sc_body(x_hbm, pk_hbm, hp_hbm,
             acc, pkb, rows,
             sem_i0, sem_i1, sem_g0, sem_g1, sem_s0, sem_s1):
    sem_i = (sem_i0, sem_i1)
    sem_g = (sem_g0, sem_g1)
    sem_s = (sem_s0, sem_s1)
    c = lax.axis_index("c")
    s = lax.axis_index("s")
    t0 = (c * NS + s) * CPT

    def issue_idx(k, b):
        pltpu.async_copy(pk_hbm.at[t0 + k], pkb.at[b], sem_i[b])

    def wait_idx(k, b):
        pltpu.make_async_copy(pk_hbm.at[t0 + k], pkb.at[b], sem_i[b]).wait()

    def issue_gather(b):
        pltpu.async_copy(x_hbm.at[pkb.at[b, 0]], rows.at[b], sem_g[b])

    def wait_gather(b):
        pltpu.make_async_copy(x_hbm.at[pkb.at[b, 0]], rows.at[b],
                              sem_g[b]).wait()

    def issue_scatter(b):
        pltpu.async_copy(rows.at[b], acc.at[pkb.at[b, 1]], sem_s[b],
                         add=True)

    def wait_scatter(b):
        pltpu.make_async_copy(rows.at[b], acc.at[pkb.at[b, 1]],
                              sem_s[b]).wait()

    def scale(b):
        def rowscale(g, rcarry):
            av = lax.bitcast_convert_type(pkb[b, 2, pl.ds(g * 16, 16)],
                                          jnp.float32)
            for t in range(16):
                a = jnp.full((16,), av[t], jnp.float32)
                for f in range(D // 16):
                    rows[b, g * 16 + t, pl.ds(f * 16, 16)] = (
                        rows[b, g * 16 + t, pl.ds(f * 16, 16)] * a)
            return rcarry

        lax.fori_loop(0, B // 16, rowscale, 0)

    issue_idx(0, 0)

    def zrow(i, carry):
        for j in range(D // 16):
            rows[0, i, pl.ds(j * 16, 16)] = jnp.zeros((16,), jnp.float32)
        return carry

    lax.fori_loop(0, B, zrow, 0)
    for q in range(ROWS_PER_TILE // B):
        pltpu.sync_copy(rows.at[0],
                        acc.at[pl.ds(s * ROWS_PER_TILE + q * B, B)])
    rem = ROWS_PER_TILE - (ROWS_PER_TILE // B) * B
    pltpu.sync_copy(
        rows.at[0, pl.ds(0, rem)],
        acc.at[pl.ds(s * ROWS_PER_TILE + ROWS_PER_TILE - rem, rem)])

    @pl.when(s == NS - 1)
    def _zero_tail():
        pltpu.sync_copy(rows.at[0, pl.ds(0, TAIL_ROWS)],
                        acc.at[pl.ds(NS * ROWS_PER_TILE, TAIL_ROWS)])

    plsc.subcore_barrier()

    wait_idx(0, 0)
    issue_gather(0)
    issue_idx(1, 1)
    wait_idx(1, 1)
    issue_gather(1)
    wait_gather(0)
    scale(0)
    issue_scatter(0)

    def loop_body(k, carry):
        for off in range(2):
            ki = k + off
            b = 1 - off
            nb = off
            wait_scatter(nb)
            issue_idx(ki + 1, nb)
            wait_gather(b)
            wait_idx(ki + 1, nb)
            issue_gather(nb)
            scale(b)
            issue_scatter(b)
        return carry

    lax.fori_loop(0, (CPT - 2) // 2, lambda i, cy: loop_body(1 + 2 * i, cy),
                  0)
    wait_scatter(0)
    wait_gather(1)
    scale(1)
    issue_scatter(1)
    wait_scatter(1)
    plsc.subcore_barrier()

    pltpu.sync_copy(acc.at[pl.ds(s * ROWS_PER_TILE, ROWS_PER_TILE)],
                    hp_hbm.at[c, pl.ds(s * ROWS_PER_TILE, ROWS_PER_TILE)])

    @pl.when(s == NS - 1)
    def _drain_tail():
        pltpu.sync_copy(acc.at[pl.ds(NS * ROWS_PER_TILE, TAIL_ROWS)],
                        hp_hbm.at[c, pl.ds(NS * ROWS_PER_TILE, TAIL_ROWS)])


def _sc_message_passing(x, pk):
    mesh = plsc.VectorSubcoreMesh(core_axis_name="c", subcore_axis_name="s")
    kern = pl.kernel(
        _sc_body,
        mesh=mesh,
        out_type=jax.ShapeDtypeStruct((NC, N, D), jnp.float32),
        scratch_types=[
            pltpu.VMEM_SHARED((N, D), jnp.float32),
            pltpu.VMEM((2, 3, B), jnp.int32),
            pltpu.VMEM((2, B, D), jnp.float32),
            pltpu.SemaphoreType.DMA,
            pltpu.SemaphoreType.DMA,
            pltpu.SemaphoreType.DMA,
            pltpu.SemaphoreType.DMA,
            pltpu.SemaphoreType.DMA,
            pltpu.SemaphoreType.DMA,
        ],
    )
    return kern(x, pk)


def _tc_body(x_ref, h0_ref, h1_ref, w1_ref, b1_ref, w2_ref, b2_ref, o_ref):
    x = x_ref[...]
    hn = h0_ref[0] + h1_ref[0]
    u = x + hn
    v = x * hn
    dn = (((1,), (1,)), ((), ()))
    y1 = lax.dot_general(u, w1_ref[...], dn,
                         preferred_element_type=jnp.float32) + b1_ref[...]
    y1 = jnp.where(y1 >= 0, y1, y1 * 0.01)
    y2 = lax.dot_general(v, w2_ref[...], dn,
                         preferred_element_type=jnp.float32) + b2_ref[...]
    y2 = jnp.where(y2 >= 0, y2, y2 * 0.01)
    o_ref[...] = y1 + y2


def _tc_dense(x, hp, W1, b1, W2, b2):
    BN = 1000
    grid = (N // BN,)
    row_spec = pl.BlockSpec((BN, D), lambda i: (i, 0))
    h0_spec = pl.BlockSpec((1, BN, D), lambda i: (0, i, 0))
    h1_spec = pl.BlockSpec((1, BN, D), lambda i: (1, i, 0))
    full_spec = pl.BlockSpec((D, D), lambda i: (0, 0))
    bias_spec = pl.BlockSpec((1, D), lambda i: (0, 0))
    return pl.pallas_call(
        _tc_body,
        grid=grid,
        in_specs=[row_spec, h0_spec, h1_spec, full_spec, bias_spec,
                  full_spec, bias_spec],
        out_specs=row_spec,
        out_shape=jax.ShapeDtypeStruct((N, D), jnp.float32),
    )(x, hp, hp, W1, b1, W2, b2)


@jax.jit
def kernel(x, edge_index, edge_attn, W1, b1, W2, b2):
    pad = EPAD - E
    src2 = jnp.concatenate(
        [edge_index[0], jnp.zeros((pad,), jnp.int32)]).reshape(NCH, 1, B)
    dst2 = jnp.concatenate(
        [edge_index[1], jnp.zeros((pad,), jnp.int32)]).reshape(NCH, 1, B)
    attn_bits = lax.bitcast_convert_type(
        jnp.concatenate([edge_attn.reshape(E), jnp.zeros((pad,),
                                                         jnp.float32)]),
        jnp.int32).reshape(NCH, 1, B)
    pk = jnp.concatenate([src2, dst2, attn_bits], axis=1)
    hp = _sc_message_passing(x, pk)
    out = _tc_dense(x, hp, W1, b1.reshape(1, D), W2, b2.reshape(1, D))
    return out


# speedup vs baseline: 1.0787x; 1.0787x over previous
import jax
import jax.numpy as jnp
from jax import lax
from jax.experimental import pallas as pl
from jax.experimental.pallas import tpu as pltpu
from jax.experimental.pallas import tpu_sc as plsc

N = 10000
E = 320000
D = 128

NC = 2
NS = 16
B = 80
G = 8
NG = 16
CPT = G * NG
EROWS = NC * NS * CPT
EPAD = EROWS * B
ROWS_PER_TILE = 624
TAIL_ROWS = N - NS * ROWS_PER_TILE


def _sc_body(x_hbm, src_hbm, dst_hbm, attn_hbm, hp_hbm,
             acc, srcb, dstb, attnb, rows, attnx,
             sem_i0, sem_i1, sem_g0, sem_g1, sem_s0, sem_s1):
    sem_i = (sem_i0, sem_i1)
    sem_g = (sem_g0, sem_g1)
    sem_s = (sem_s0, sem_s1)
    c = lax.axis_index("c")
    s = lax.axis_index("s")
    t0r = (c * NS + s) * CPT

    def issue_idx(g, bs):
        r0 = t0r + g * G
        pltpu.async_copy(src_hbm.at[pl.ds(r0, G)], srcb.at[bs], sem_i[bs])
        pltpu.async_copy(dst_hbm.at[pl.ds(r0, G)], dstb.at[bs], sem_i[bs])
        pltpu.async_copy(attn_hbm.at[pl.ds(r0, G)], attnb.at[bs],
                         sem_i[bs])

    def wait_idx(g, bs):
        r0 = t0r + g * G
        pltpu.make_async_copy(src_hbm.at[pl.ds(r0, G)], srcb.at[bs],
                              sem_i[bs]).wait()
        pltpu.make_async_copy(dst_hbm.at[pl.ds(r0, G)], dstb.at[bs],
                              sem_i[bs]).wait()
        pltpu.make_async_copy(attn_hbm.at[pl.ds(r0, G)], attnb.at[bs],
                              sem_i[bs]).wait()

    def issue_gather(gs, j, b):
        pltpu.async_copy(x_hbm.at[srcb.at[gs, j]], rows.at[b], sem_g[b])

    def wait_gather(gs, j, b):
        pltpu.make_async_copy(x_hbm.at[srcb.at[gs, j]], rows.at[b],
                              sem_g[b]).wait()

    def issue_scatter(gs, j, b):
        pltpu.async_copy(rows.at[b], acc.at[dstb.at[gs, j]], sem_s[b],
                         add=True)

    def wait_scatter(gs, j, b):
        pltpu.make_async_copy(rows.at[b], acc.at[dstb.at[gs, j]],
                              sem_s[b]).wait()

    def scale(gs, j, b):
        def exprow(g5, cy):
            av = attnb[gs, j, pl.ds(g5 * 16, 16)]
            for t in range(16):
                attnx[pl.ds((g5 * 16 + t) * 16, 16)] = jnp.full(
                    (16,), av[t], jnp.float32)
            return cy

        lax.fori_loop(0, B // 16, exprow, 0)

        def mulrow(i, cy):
            a = attnx[pl.ds(i * 16, 16)]
            for f in range(D // 16):
                rows[b, i, pl.ds(f * 16, 16)] = (
                    rows[b, i, pl.ds(f * 16, 16)] * a)
            return cy

        lax.fori_loop(0, B, mulrow, 0)

    def sub_iter(g, gs, j, last_group=False):
        b = j & 1
        nb = 1 - b
        if j == 0:
            wait_scatter(1 - gs, G - 1, nb)
            if not last_group:
                issue_idx(g + 1, 1 - gs)
        else:
            wait_scatter(gs, j - 1, nb)
        if j == G - 1:
            if not last_group:
                wait_idx(g + 1, 1 - gs)
                issue_gather(1 - gs, 0, nb)
        else:
            issue_gather(gs, j + 1, nb)
        wait_gather(gs, j, b)
        scale(gs, j, b)
        issue_scatter(gs, j, b)

    issue_idx(0, 0)

    def zrow(i, carry):
        for j in range(D // 16):
            rows[0, i, pl.ds(j * 16, 16)] = jnp.zeros((16,), jnp.float32)
        return carry

    lax.fori_loop(0, B, zrow, 0)
    for q in range(ROWS_PER_TILE // B):
        pltpu.sync_copy(rows.at[0],
                        acc.at[pl.ds(s * ROWS_PER_TILE + q * B, B)])
    rem = ROWS_PER_TILE - (ROWS_PER_TILE // B) * B
    pltpu.sync_copy(
        rows.at[0, pl.ds(0, rem)],
        acc.at[pl.ds(s * ROWS_PER_TILE + ROWS_PER_TILE - rem, rem)])

    @pl.when(s == NS - 1)
    def _zero_tail():
        pltpu.sync_copy(rows.at[0, pl.ds(0, TAIL_ROWS)],
                        acc.at[pl.ds(NS * ROWS_PER_TILE, TAIL_ROWS)])

    plsc.subcore_barrier()

    issue_idx(1, 1)
    wait_idx(0, 0)
    issue_gather(0, 0, 0)
    issue_gather(0, 1, 1)
    wait_gather(0, 0, 0)
    scale(0, 0, 0)
    issue_scatter(0, 0, 0)
    for j in range(1, G):
        sub_iter(0, 0, j)

    def pair_body(m, carry):
        for go, gs in ((1, 1), (2, 0)):
            g = 2 * m + go
            for j in range(G):
                sub_iter(g, gs, j)
        return carry

    lax.fori_loop(0, (NG - 2) // 2, pair_body, 0)

    for j in range(G):
        sub_iter(NG - 1, NG % 2 ^ 1, j, last_group=True)
    wait_scatter(1, G - 1, (G - 1) & 1)
    plsc.subcore_barrier()

    pltpu.sync_copy(acc.at[pl.ds(s * ROWS_PER_TILE, ROWS_PER_TILE)],
                    hp_hbm.at[c, pl.ds(s * ROWS_PER_TILE, ROWS_PER_TILE)])

    @pl.when(s == NS - 1)
    def _drain_tail():
        pltpu.sync_copy(acc.at[pl.ds(NS * ROWS_PER_TILE, TAIL_ROWS)],
                        hp_hbm.at[c, pl.ds(NS * ROWS_PER_TILE, TAIL_ROWS)])


def _sc_message_passing(x, src2, dst2, attn2):
    mesh = plsc.VectorSubcoreMesh(core_axis_name="c", subcore_axis_name="s")
    kern = pl.kernel(
        _sc_body,
        mesh=mesh,
        out_type=jax.ShapeDtypeStruct((NC, N, D), jnp.float32),
        scratch_types=[
            pltpu.VMEM_SHARED((N, D), jnp.float32),
            pltpu.VMEM((2, G, B), jnp.int32),
            pltpu.VMEM((2, G, B), jnp.int32),
            pltpu.VMEM((2, G, B), jnp.float32),
            pltpu.VMEM((2, B, D), jnp.float32),
            pltpu.VMEM((B * 16,), jnp.float32),
            pltpu.SemaphoreType.DMA,
            pltpu.SemaphoreType.DMA,
            pltpu.SemaphoreType.DMA,
            pltpu.SemaphoreType.DMA,
            pltpu.SemaphoreType.DMA,
            pltpu.SemaphoreType.DMA,
        ],
    )
    return kern(x, src2, dst2, attn2)


def _tc_body(x_ref, h0_ref, h1_ref, w1_ref, b1_ref, w2_ref, b2_ref, o_ref):
    x = x_ref[...]
    hn = h0_ref[0] + h1_ref[0]
    u = x + hn
    v = x * hn
    dn = (((1,), (1,)), ((), ()))
    y1 = lax.dot_general(u, w1_ref[...], dn,
                         preferred_element_type=jnp.float32) + b1_ref[...]
    y1 = jnp.where(y1 >= 0, y1, y1 * 0.01)
    y2 = lax.dot_general(v, w2_ref[...], dn,
                         preferred_element_type=jnp.float32) + b2_ref[...]
    y2 = jnp.where(y2 >= 0, y2, y2 * 0.01)
    o_ref[...] = y1 + y2


def _tc_dense(x, hp, W1, b1, W2, b2):
    BN = 1000
    grid = (N // BN,)
    row_spec = pl.BlockSpec((BN, D), lambda i: (i, 0))
    h0_spec = pl.BlockSpec((1, BN, D), lambda i: (0, i, 0))
    h1_spec = pl.BlockSpec((1, BN, D), lambda i: (1, i, 0))
    full_spec = pl.BlockSpec((D, D), lambda i: (0, 0))
    bias_spec = pl.BlockSpec((1, D), lambda i: (0, 0))
    return pl.pallas_call(
        _tc_body,
        grid=grid,
        in_specs=[row_spec, h0_spec, h1_spec, full_spec, bias_spec,
                  full_spec, bias_spec],
        out_specs=row_spec,
        out_shape=jax.ShapeDtypeStruct((N, D), jnp.float32),
    )(x, hp, hp, W1, b1, W2, b2)


@jax.jit
def kernel(x, edge_index, edge_attn, W1, b1, W2, b2):
    pad = EPAD - E
    src2 = jnp.concatenate(
        [edge_index[0], jnp.zeros((pad,), jnp.int32)]).reshape(EROWS, B)
    dst2 = jnp.concatenate(
        [edge_index[1], jnp.zeros((pad,), jnp.int32)]).reshape(EROWS, B)
    attn2 = jnp.concatenate(
        [edge_attn.reshape(E), jnp.zeros((pad,), jnp.float32)]
    ).reshape(EROWS, B)
    hp = _sc_message_passing(x, src2, dst2, attn2)
    out = _tc_dense(x, hp, W1, b1.reshape(1, D), W2, b2.reshape(1, D))
    return out


# speedup vs baseline: 2.4977x; 2.3156x over previous
import jax
import jax.numpy as jnp
from jax import lax
from jax.experimental import pallas as pl
from jax.experimental.pallas import tpu as pltpu
from jax.experimental.pallas import tpu_sc as plsc

N = 10000
E = 320000
D = 128

NC = 2
NS = 16
B = 80
EDGES_PER_TILE = E // (NC * NS)
CHUNKS = EDGES_PER_TILE // B
ROWS_PER_TILE = 624
TAIL_ROWS = N - NS * ROWS_PER_TILE


def _sc_body(x_hbm, src_hbm, dst_hbm, attn_hbm, hp_hbm,
             acc, srcb, dstb, attnb, rows, dummy,
             sem_i0, sem_i1, sem_g0, sem_g1, sem_s0, sem_s1):
    sem_i = (sem_i0, sem_i1)
    sem_g = (sem_g0, sem_g1)
    sem_s = (sem_s0, sem_s1)
    c = lax.axis_index("c")
    s = lax.axis_index("s")
    base0 = (c * NS + s) * EDGES_PER_TILE

    def issue_idx(ki, b):
        base = base0 + ki * B
        pltpu.async_copy(src_hbm.at[pl.ds(base, B)], srcb.at[b], sem_i[b])
        pltpu.async_copy(dst_hbm.at[pl.ds(base, B)], dstb.at[b], sem_i[b])
        pltpu.async_copy(attn_hbm.at[pl.ds(base, 96)], attnb.at[b],
                         sem_i[b])

    def wait_idx(ki, b):
        pltpu.make_async_copy(x_hbm.at[pl.ds(0, 2)], dummy.at[b],
                              sem_i[b]).wait()

    def issue_gather(b):
        pltpu.async_copy(x_hbm.at[srcb.at[b]], rows.at[b], sem_g[b])

    def wait_gather(b):
        pltpu.make_async_copy(x_hbm.at[srcb.at[b]], rows.at[b],
                              sem_g[b]).wait()

    def issue_scatter(b):
        pltpu.async_copy(rows.at[b], acc.at[dstb.at[b]], sem_s[b], add=True)

    def wait_scatter(b):
        pltpu.make_async_copy(rows.at[b], acc.at[dstb.at[b]],
                              sem_s[b]).wait()

    def scale(b):
        def rowscale(g, rcarry):
            av = attnb[b, pl.ds(g * 16, 16)]
            for t in range(16):
                i = g * 16 + t
                a = jnp.full((16,), av[t], jnp.float32)
                for f in range(D // 16):
                    rows[b, i, pl.ds(f * 16, 16)] = (
                        rows[b, i, pl.ds(f * 16, 16)] * a)
            return rcarry

        lax.fori_loop(0, B // 16, rowscale, 0)

    issue_idx(0, 0)

    def zrow(i, carry):
        for j in range(D // 16):
            rows[0, i, pl.ds(j * 16, 16)] = jnp.zeros((16,), jnp.float32)
        return carry

    lax.fori_loop(0, B, zrow, 0)
    for q in range(ROWS_PER_TILE // B):
        pltpu.sync_copy(rows.at[0],
                        acc.at[pl.ds(s * ROWS_PER_TILE + q * B, B)])
    rem = ROWS_PER_TILE - (ROWS_PER_TILE // B) * B
    pltpu.sync_copy(
        rows.at[0, pl.ds(0, rem)],
        acc.at[pl.ds(s * ROWS_PER_TILE + ROWS_PER_TILE - rem, rem)])

    @pl.when(s == NS - 1)
    def _zero_tail():
        pltpu.sync_copy(rows.at[0, pl.ds(0, TAIL_ROWS)],
                        acc.at[pl.ds(NS * ROWS_PER_TILE, TAIL_ROWS)])

    plsc.subcore_barrier()

    wait_idx(0, 0)
    issue_gather(0)
    issue_idx(1, 1)
    wait_idx(1, 1)
    issue_gather(1)
    wait_gather(0)
    scale(0)
    issue_scatter(0)

    def loop_body(k, carry):
        for off in range(2):
            ki = k + off
            b = 1 - off
            nb = off
            knext = jnp.minimum(ki + 1, CHUNKS - 1)
            wait_scatter(nb)
            issue_idx(knext, nb)
            wait_gather(b)
            wait_idx(knext, nb)
            issue_gather(nb)
            scale(b)
            issue_scatter(b)
        return carry

    lax.fori_loop(0, (CHUNKS - 1) // 2,
                  lambda i, cy: loop_body(1 + 2 * i, cy), 0)
    wait_gather(1)
    wait_scatter(0)
    plsc.subcore_barrier()

    pltpu.sync_copy(acc.at[pl.ds(s * ROWS_PER_TILE, ROWS_PER_TILE)],
                    hp_hbm.at[c, pl.ds(s * ROWS_PER_TILE, ROWS_PER_TILE)])

    @pl.when(s == NS - 1)
    def _drain_tail():
        pltpu.sync_copy(acc.at[pl.ds(NS * ROWS_PER_TILE, TAIL_ROWS)],
                        hp_hbm.at[c, pl.ds(NS * ROWS_PER_TILE, TAIL_ROWS)])


def _sc_message_passing(x, src, dst, attn):
    mesh = plsc.VectorSubcoreMesh(core_axis_name="c", subcore_axis_name="s")
    kern = pl.kernel(
        _sc_body,
        mesh=mesh,
        out_type=jax.ShapeDtypeStruct((NC, N, D), jnp.float32),
        scratch_types=[
            pltpu.VMEM_SHARED((N, D), jnp.float32),
            pltpu.VMEM((2, B), jnp.int32),
            pltpu.VMEM((2, B), jnp.int32),
            pltpu.VMEM((2, 96), jnp.float32),
            pltpu.VMEM((2, B, D), jnp.float32),
            pltpu.VMEM((2, 2, D), jnp.float32),
            pltpu.SemaphoreType.DMA,
            pltpu.SemaphoreType.DMA,
            pltpu.SemaphoreType.DMA,
            pltpu.SemaphoreType.DMA,
            pltpu.SemaphoreType.DMA,
            pltpu.SemaphoreType.DMA,
        ],
    )
    return kern(x, src, dst, attn)


def _tc_body(x_ref, h0_ref, h1_ref, w1_ref, b1_ref, w2_ref, b2_ref, o_ref):
    x = x_ref[...]
    hn = h0_ref[0] + h1_ref[0]
    u = x + hn
    v = x * hn
    dn = (((1,), (1,)), ((), ()))
    y1 = lax.dot_general(u, w1_ref[...], dn,
                         preferred_element_type=jnp.float32) + b1_ref[...]
    y1 = jnp.where(y1 >= 0, y1, y1 * 0.01)
    y2 = lax.dot_general(v, w2_ref[...], dn,
                         preferred_element_type=jnp.float32) + b2_ref[...]
    y2 = jnp.where(y2 >= 0, y2, y2 * 0.01)
    o_ref[...] = y1 + y2


def _tc_dense(x, hp, W1, b1, W2, b2):
    BN = 2000
    grid = (N // BN,)
    row_spec = pl.BlockSpec((BN, D), lambda i: (i, 0))
    h0_spec = pl.BlockSpec((1, BN, D), lambda i: (0, i, 0))
    h1_spec = pl.BlockSpec((1, BN, D), lambda i: (1, i, 0))
    full_spec = pl.BlockSpec((D, D), lambda i: (0, 0))
    bias_spec = pl.BlockSpec((1, D), lambda i: (0, 0))
    return pl.pallas_call(
        _tc_body,
        grid=grid,
        in_specs=[row_spec, h0_spec, h1_spec, full_spec, bias_spec,
                  full_spec, bias_spec],
        out_specs=row_spec,
        out_shape=jax.ShapeDtypeStruct((N, D), jnp.float32),
    )(x, hp, hp, W1, b1, W2, b2)


@jax.jit
def kernel(x, edge_index, edge_attn, W1, b1, W2, b2):
    attn_p = jnp.concatenate(
        [edge_attn.reshape(E), jnp.zeros((16,), jnp.float32)])
    hp = _sc_message_passing(x, edge_index[0], edge_index[1], attn_p)
    out = _tc_dense(x, hp, W1, b1.reshape(1, D), W2, b2.reshape(1, D))
    return out


# speedup vs baseline: 3.0278x; 1.2122x over previous
import jax
import jax.numpy as jnp
from jax import lax
from jax.experimental import pallas as pl
from jax.experimental.pallas import tpu as pltpu
from jax.experimental.pallas import tpu_sc as plsc

N = 10000
E = 320000
D = 128

NC = 2
NS = 16
B = 80
EDGES_PER_TILE = E // (NC * NS)
CHUNKS = EDGES_PER_TILE // B
ROWS_PER_TILE = 624
TAIL_ROWS = N - NS * ROWS_PER_TILE


def _sc_body(x_hbm, src_hbm, dst_hbm, attn_hbm, hp_hbm,
             acc, srcb, dstb, attnb, rows, dummy,
             sem_i0, sem_i1, sem_i2, sem_i3,
             sem_g0, sem_g1, sem_s0, sem_s1):
    sem_i = (sem_i0, sem_i1, sem_i2, sem_i3)
    sem_g = (sem_g0, sem_g1)
    sem_s = (sem_s0, sem_s1)
    c = lax.axis_index("c")
    s = lax.axis_index("s")
    base0 = (c * NS + s) * EDGES_PER_TILE

    def issue_idx(ki, b):
        base = base0 + ki * B
        pltpu.async_copy(src_hbm.at[pl.ds(base, B)], srcb.at[b], sem_i[b])
        pltpu.async_copy(dst_hbm.at[pl.ds(base, B)], dstb.at[b], sem_i[b])
        pltpu.async_copy(attn_hbm.at[pl.ds(base, 96)], attnb.at[b],
                         sem_i[b])

    def wait_idx(b):
        pltpu.make_async_copy(x_hbm.at[pl.ds(0, 2)], dummy.at[0],
                              sem_i[b]).wait()

    def issue_gather(ib, rb):
        pltpu.async_copy(x_hbm.at[srcb.at[ib]], rows.at[rb], sem_g[rb])

    def wait_gather(ib, rb):
        pltpu.make_async_copy(x_hbm.at[srcb.at[ib]], rows.at[rb],
                              sem_g[rb]).wait()

    def issue_scatter(ib, rb):
        pltpu.async_copy(rows.at[rb], acc.at[dstb.at[ib]], sem_s[rb],
                         add=True)

    def wait_scatter(ib, rb):
        pltpu.make_async_copy(rows.at[rb], acc.at[dstb.at[ib]],
                              sem_s[rb]).wait()

    def scale(ib, rb):
        def rowscale(g, rcarry):
            av = attnb[ib, pl.ds(g * 16, 16)]
            for t in range(16):
                i = g * 16 + t
                a = jnp.full((16,), av[t], jnp.float32)
                for f in range(D // 16):
                    rows[rb, i, pl.ds(f * 16, 16)] = (
                        rows[rb, i, pl.ds(f * 16, 16)] * a)
            return rcarry

        lax.fori_loop(0, B // 16, rowscale, 0)

    issue_idx(0, 0)

    def zrow(i, carry):
        for j in range(D // 16):
            rows[0, i, pl.ds(j * 16, 16)] = jnp.zeros((16,), jnp.float32)
        return carry

    lax.fori_loop(0, B, zrow, 0)
    for q in range(ROWS_PER_TILE // B):
        pltpu.sync_copy(rows.at[0],
                        acc.at[pl.ds(s * ROWS_PER_TILE + q * B, B)])
    rem = ROWS_PER_TILE - (ROWS_PER_TILE // B) * B
    pltpu.sync_copy(
        rows.at[0, pl.ds(0, rem)],
        acc.at[pl.ds(s * ROWS_PER_TILE + ROWS_PER_TILE - rem, rem)])

    @pl.when(s == NS - 1)
    def _zero_tail():
        pltpu.sync_copy(rows.at[0, pl.ds(0, TAIL_ROWS)],
                        acc.at[pl.ds(NS * ROWS_PER_TILE, TAIL_ROWS)])

    plsc.subcore_barrier()

    issue_idx(1, 1)
    issue_idx(2, 2)
    issue_idx(3, 3)
    wait_idx(0)
    issue_gather(0, 0)
    wait_idx(1)
    issue_gather(1, 1)
    wait_gather(0, 0)
    scale(0, 0)
    issue_scatter(0, 0)

    def loop_body(k, carry):
        for off in range(4):
            ki = k + off
            rb = (1 + off) & 1
            nrb = 1 - rb
            cur = (1 + off) & 3
            nxt = (2 + off) & 3
            fut = off & 3
            wait_scatter(fut, nrb)
            issue_idx(jnp.minimum(ki + 3, CHUNKS - 1), fut)
            wait_idx(nxt)
            issue_gather(nxt, nrb)
            wait_gather(cur, rb)
            scale(cur, rb)
            issue_scatter(cur, rb)
        return carry

    lax.fori_loop(0, (CHUNKS - 1) // 4,
                  lambda i, cy: loop_body(1 + 4 * i, cy), 0)
    wait_gather(1, 1)
    wait_scatter(0, 0)
    wait_idx(2)
    wait_idx(3)
    plsc.subcore_barrier()

    pltpu.sync_copy(acc.at[pl.ds(s * ROWS_PER_TILE, ROWS_PER_TILE)],
                    hp_hbm.at[c, pl.ds(s * ROWS_PER_TILE, ROWS_PER_TILE)])

    @pl.when(s == NS - 1)
    def _drain_tail():
        pltpu.sync_copy(acc.at[pl.ds(NS * ROWS_PER_TILE, TAIL_ROWS)],
                        hp_hbm.at[c, pl.ds(NS * ROWS_PER_TILE, TAIL_ROWS)])


def _sc_message_passing(x, src, dst, attn):
    mesh = plsc.VectorSubcoreMesh(core_axis_name="c", subcore_axis_name="s")
    kern = pl.kernel(
        _sc_body,
        mesh=mesh,
        out_type=jax.ShapeDtypeStruct((NC, N, D), jnp.float32),
        scratch_types=[
            pltpu.VMEM_SHARED((N, D), jnp.float32),
            pltpu.VMEM((4, B), jnp.int32),
            pltpu.VMEM((4, B), jnp.int32),
            pltpu.VMEM((4, 96), jnp.float32),
            pltpu.VMEM((2, B, D), jnp.float32),
            pltpu.VMEM((2, 2, D), jnp.float32),
            pltpu.SemaphoreType.DMA,
            pltpu.SemaphoreType.DMA,
            pltpu.SemaphoreType.DMA,
            pltpu.SemaphoreType.DMA,
            pltpu.SemaphoreType.DMA,
            pltpu.SemaphoreType.DMA,
            pltpu.SemaphoreType.DMA,
            pltpu.SemaphoreType.DMA,
        ],
    )
    return kern(x, src, dst, attn)


def _tc_body(x_ref, h0_ref, h1_ref, w1_ref, b1_ref, w2_ref, b2_ref, o_ref):
    x = x_ref[...]
    hn = h0_ref[0] + h1_ref[0]
    u = x + hn
    v = x * hn
    dn = (((1,), (1,)), ((), ()))
    y1 = lax.dot_general(u, w1_ref[...], dn,
                         preferred_element_type=jnp.float32) + b1_ref[...]
    y1 = jnp.where(y1 >= 0, y1, y1 * 0.01)
    y2 = lax.dot_general(v, w2_ref[...], dn,
                         preferred_element_type=jnp.float32) + b2_ref[...]
    y2 = jnp.where(y2 >= 0, y2, y2 * 0.01)
    o_ref[...] = y1 + y2


def _tc_dense(x, hp, W1, b1, W2, b2):
    BN = 2000
    grid = (N // BN,)
    row_spec = pl.BlockSpec((BN, D), lambda i: (i, 0))
    h0_spec = pl.BlockSpec((1, BN, D), lambda i: (0, i, 0))
    h1_spec = pl.BlockSpec((1, BN, D), lambda i: (1, i, 0))
    full_spec = pl.BlockSpec((D, D), lambda i: (0, 0))
    bias_spec = pl.BlockSpec((1, D), lambda i: (0, 0))
    return pl.pallas_call(
        _tc_body,
        grid=grid,
        in_specs=[row_spec, h0_spec, h1_spec, full_spec, bias_spec,
                  full_spec, bias_spec],
        out_specs=row_spec,
        out_shape=jax.ShapeDtypeStruct((N, D), jnp.float32),
    )(x, hp, hp, W1, b1, W2, b2)


@jax.jit
def kernel(x, edge_index, edge_attn, W1, b1, W2, b2):
    attn_p = jnp.concatenate(
        [edge_attn.reshape(E), jnp.zeros((16,), jnp.float32)])
    hp = _sc_message_passing(x, edge_index[0], edge_index[1], attn_p)
    out = _tc_dense(x, hp, W1, b1.reshape(1, D), W2, b2.reshape(1, D))
    return out
